# Initial kernel scaffold; baseline (speedup 1.0000x reference)
import jax
import jax.numpy as jnp
from jax.experimental import pallas as pl


def kernel(all_codes_embs, input_ids, wqkvT, bqkv, wo, w1, w2, vec):
    raise NotImplementedError("write your pallas kernel here")


# speedup vs baseline: 1.6332x; 1.6332x over previous
import jax
import jax.numpy as jnp
from jax import lax
from jax.experimental import pallas as pl
from jax.experimental.pallas import tpu as pltpu
from functools import partial


def _encoder_kernel(nh, hd, n_layers, L, eps, eb, group,
                    x_ref, keep_ref, pw_ref,
                    wqkvT_ref, bqkv_ref, wo_ref, w1_ref, w2_ref, vec_ref,
                    o_ref):
    D = nh * hd
    R = x_ref.shape[0]
    G = R // group

    h = x_ref[...]
    key_ok = keep_ref[0] > 0.0

    r_ev = lax.broadcasted_iota(jnp.int32, (group, group), 0) // L
    c_ev = lax.broadcasted_iota(jnp.int32, (group, group), 1) // L
    diag = r_ev == c_ev
    biases = []
    for g in range(G):
        okg = key_ok[:, g * group:(g + 1) * group]
        biases.append(jnp.where(diag & okg, 0.0, -1e30))

    def layer_norm(y, gain, bias):
        mu = jnp.mean(y, axis=-1, keepdims=True)
        var = jnp.mean((y - mu) ** 2, axis=-1, keepdims=True)
        return (y - mu) * lax.rsqrt(var + eps) * gain + bias

    for l in range(n_layers):
        wqkvT = wqkvT_ref[l]
        bqkv = bqkv_ref[l]
        wo = wo_ref[l]
        w1 = w1_ref[l]
        w2 = w2_ref[l]
        vec = vec_ref[l]
        b1 = vec[0:1, :]
        b_o = vec[1:2, :D]
        g1 = vec[2:3, :D]
        be1 = vec[3:4, :D]
        b2 = vec[4:5, :D]
        g2 = vec[5:6, :D]
        be2 = vec[6:7, :D]

        hb = h.astype(jnp.bfloat16)
        qkvT = lax.dot_general(wqkvT, hb, (((1,), (1,)), ((), ())),
                               preferred_element_type=jnp.float32) + bqkv
        qkv3 = qkvT.reshape(3 * nh, hd, R)
        qT = qkv3[0:nh]
        kT = qkv3[nh:2 * nh].astype(jnp.bfloat16)
        vT = qkv3[2 * nh:3 * nh].astype(jnp.bfloat16)
        q = pltpu.einshape("hdr->hrd", qT).astype(jnp.bfloat16)

        ctxT_parts = []
        for g in range(G):
            lo = g * group
            qg = q[:, lo:lo + group, :]
            kTg = kT[:, :, lo:lo + group]
            vTg = vT[:, :, lo:lo + group]
            sg = jnp.einsum("hqd,hdk->hqk", qg, kTg,
                            preferred_element_type=jnp.float32)
            sg = sg + biases[g]
            sg = sg - jnp.max(sg, axis=-1, keepdims=True)
            pg = jnp.exp(sg)
            pg = pg * pl.reciprocal(jnp.sum(pg, axis=-1, keepdims=True),
                                    approx=True)
            ctxT_parts.append(
                jnp.einsum("hdk,hqk->hdq", vTg, pg.astype(jnp.bfloat16),
                           preferred_element_type=jnp.float32))
        ctxT = ctxT_parts[0] if G == 1 else jnp.concatenate(ctxT_parts, axis=2)
        ctx = jnp.transpose(ctxT.reshape(D, R))
        attn = jnp.dot(ctx.astype(jnp.bfloat16), wo,
                       preferred_element_type=jnp.float32) + b_o

        y = layer_norm(h + attn, g1, be1)
        yb = y.astype(jnp.bfloat16)
        ff = jnp.maximum(jnp.dot(yb, w1, preferred_element_type=jnp.float32) + b1,
                         0.0)
        ff = jnp.dot(ff.astype(jnp.bfloat16), w2,
                     preferred_element_type=jnp.float32) + b2
        h = layer_norm(y + ff, g2, be2)

    pw = pw_ref[...]
    h3 = h.reshape(eb, L, D)
    o_ref[...] = jnp.sum(h3 * pw[:, :, None], axis=1)


def _run_encoder(x_flat, keep, pool_w, params, n_heads):
    wqkvT, bqkv, wo, w1, w2, vec = params
    NL, D = x_flat.shape
    N, L = keep.shape
    n_layers = wqkvT.shape[0]
    hd = D // n_heads

    eb = 0
    for c in range(1, min(N, max(1, 512 // L)) + 1):
        if N % c == 0 and (c * L) % 128 == 0:
            eb = c
    if eb == 0:
        for c in range(1, N + 1):
            if N % c == 0 and (c * L) % 8 == 0:
                eb = c
    R = eb * L
    group = 128 if R % 128 == 0 else R
    n_blocks = N // eb

    keep_blk = keep.astype(jnp.float32).reshape(n_blocks, 1, R)
    pool_w = pool_w.astype(jnp.float32)

    body = partial(_encoder_kernel, n_heads, hd, n_layers, L, 1e-5, eb, group)

    def whole(arr):
        nd = arr.ndim
        return pl.BlockSpec(arr.shape, lambda i, _nd=nd: (0,) * _nd)

    return pl.pallas_call(
        body,
        out_shape=jax.ShapeDtypeStruct((N, D), jnp.float32),
        grid=(n_blocks,),
        in_specs=[
            pl.BlockSpec((R, D), lambda i: (i, 0)),
            pl.BlockSpec((1, 1, R), lambda i: (i, 0, 0)),
            pl.BlockSpec((eb, L), lambda i: (i, 0)),
            whole(wqkvT), whole(bqkv), whole(wo),
            whole(w1), whole(w2), whole(vec),
        ],
        out_specs=pl.BlockSpec((eb, D), lambda i: (i, 0)),
        compiler_params=pltpu.CompilerParams(
            dimension_semantics=("parallel",),
            vmem_limit_bytes=64 * 1024 * 1024,
        ),
    )(x_flat, keep_blk, pool_w, wqkvT, bqkv, wo, w1, w2, vec)


def kernel(all_codes_embs, input_ids, wqkvT, bqkv, wo, w1, w2, vec):
    B, S, L = input_ids.shape
    N = B * S
    D = all_codes_embs.shape[-1]

    ids2 = input_ids.reshape(N, L)
    row_zero = jnp.all(ids2 == 0, axis=-1)
    pad = (ids2 == 0) ^ row_zero[:, None]
    keep = jnp.logical_not(pad).astype(jnp.float32)
    ev = jnp.logical_not(row_zero).astype(jnp.float32)[:, None]
    pool_w = keep * ev / keep.sum(axis=-1, keepdims=True)

    x_flat = all_codes_embs.reshape(N * L, D).astype(jnp.float32)
    pooled = _run_encoder(x_flat, keep, pool_w,
                          (wqkvT, bqkv, wo, w1, w2, vec), 8)
    return pooled.reshape(B, S, D)


# speedup vs baseline: 2.6731x; 1.6368x over previous
import jax
import jax.numpy as jnp
from jax import lax
from jax.experimental import pallas as pl
from jax.experimental.pallas import tpu as pltpu
from functools import partial


def _encoder_kernel(nh, hd, n_layers, L, eps, eb, group,
                    x_ref, keep_ref, pw_ref, diag_ref,
                    wqkvT_ref, bqkv_ref, wo_ref, w1_ref, w2_ref, vec_ref,
                    o_ref):
    D = nh * hd
    R = x_ref.shape[0]
    G = R // group

    h = x_ref[...]
    key_ok = keep_ref[0] > 0.0

    diagbias = diag_ref[...]
    biases = []
    for g in range(G):
        okg = key_ok[:, g * group:(g + 1) * group]
        biases.append(diagbias + jnp.where(okg, 0.0, -1e30))

    def layer_norm(y, gain, bias):
        mu = jnp.mean(y, axis=-1, keepdims=True)
        var = jnp.mean((y - mu) ** 2, axis=-1, keepdims=True)
        return (y - mu) * lax.rsqrt(var + eps) * gain + bias

    for l in range(n_layers):
        wqkvT = wqkvT_ref[l]
        bqkv = bqkv_ref[l]
        wo = wo_ref[l]
        w1 = w1_ref[l]
        w2 = w2_ref[l]
        vec = vec_ref[l]
        b1 = vec[0:1, :]
        b_o = vec[1:2, :D]
        g1 = vec[2:3, :D]
        be1 = vec[3:4, :D]
        b2 = vec[4:5, :D]
        g2 = vec[5:6, :D]
        be2 = vec[6:7, :D]

        hb = h.astype(jnp.bfloat16)
        qkvT = lax.dot_general(wqkvT, hb, (((1,), (1,)), ((), ())),
                               preferred_element_type=jnp.float32) + bqkv
        qkv3 = qkvT.reshape(3 * nh, hd, R)
        qT = qkv3[0:nh]
        kT = qkv3[nh:2 * nh].astype(jnp.bfloat16)
        vT = qkv3[2 * nh:3 * nh].astype(jnp.bfloat16)
        vTa = jnp.concatenate(
            [vT, jnp.ones((nh, 16, R), jnp.bfloat16)], axis=1)
        q = pltpu.einshape("hdr->hrd", qT).astype(jnp.bfloat16)

        ctxT_parts = []
        for g in range(G):
            lo = g * group
            qg = q[:, lo:lo + group, :]
            kTg = kT[:, :, lo:lo + group]
            vTg = vTa[:, :, lo:lo + group]
            sg = jnp.einsum("hqd,hdk->hqk", qg, kTg,
                            preferred_element_type=jnp.float32)
            pg = jnp.exp(jnp.minimum(sg + biases[g], 60.0))
            cd = jnp.einsum("hdk,hqk->hdq", vTg, pg.astype(jnp.bfloat16),
                            preferred_element_type=jnp.float32)
            denom = cd[:, hd:hd + 1, :]
            ctxT_parts.append(
                cd[:, :hd, :] * pl.reciprocal(denom + 1e-30, approx=True))
        ctxT = ctxT_parts[0] if G == 1 else jnp.concatenate(ctxT_parts, axis=2)
        ctx = jnp.transpose(ctxT.reshape(D, R))
        attn = jnp.dot(ctx.astype(jnp.bfloat16), wo,
                       preferred_element_type=jnp.float32) + b_o

        y = layer_norm(h + attn, g1, be1)
        yb = y.astype(jnp.bfloat16)
        ff = jnp.maximum(jnp.dot(yb, w1, preferred_element_type=jnp.float32) + b1,
                         0.0)
        ff = jnp.dot(ff.astype(jnp.bfloat16), w2,
                     preferred_element_type=jnp.float32) + b2
        h = layer_norm(y + ff, g2, be2)

    pw = pw_ref[...]
    h3 = h.reshape(eb, L, D)
    o_ref[...] = jnp.sum(h3 * pw[:, :, None], axis=1)


def _run_encoder(x_flat, keep, pool_w, params, n_heads):
    wqkvT, bqkv, wo, w1, w2, vec = params
    NL, D = x_flat.shape
    N, L = keep.shape
    n_layers = wqkvT.shape[0]
    hd = D // n_heads

    eb = 0
    for c in range(1, min(N, max(1, 1024 // L)) + 1):
        if N % c == 0 and (c * L) % 128 == 0:
            eb = c
    if eb == 0:
        for c in range(1, N + 1):
            if N % c == 0 and (c * L) % 8 == 0:
                eb = c
    R = eb * L
    group = 128 if R % 128 == 0 else R
    n_blocks = N // eb

    keep_blk = keep.astype(jnp.float32).reshape(n_blocks, 1, R)
    pool_w = pool_w.astype(jnp.float32)

    r_ev = jnp.arange(group, dtype=jnp.int32)[:, None] // L
    c_ev = jnp.arange(group, dtype=jnp.int32)[None, :] // L
    diagbias = jnp.where(r_ev == c_ev, 0.0, -1e30).astype(jnp.float32)

    body = partial(_encoder_kernel, n_heads, hd, n_layers, L, 1e-5, eb, group)

    def whole(arr):
        nd = arr.ndim
        return pl.BlockSpec(arr.shape, lambda i, _nd=nd: (0,) * _nd)

    return pl.pallas_call(
        body,
        out_shape=jax.ShapeDtypeStruct((N, D), jnp.float32),
        grid=(n_blocks,),
        in_specs=[
            pl.BlockSpec((R, D), lambda i: (i, 0)),
            pl.BlockSpec((1, 1, R), lambda i: (i, 0, 0)),
            pl.BlockSpec((eb, L), lambda i: (i, 0)),
            whole(diagbias),
            whole(wqkvT), whole(bqkv), whole(wo),
            whole(w1), whole(w2), whole(vec),
        ],
        out_specs=pl.BlockSpec((eb, D), lambda i: (i, 0)),
        compiler_params=pltpu.CompilerParams(
            dimension_semantics=("parallel",),
            vmem_limit_bytes=64 * 1024 * 1024,
        ),
    )(x_flat, keep_blk, pool_w, diagbias, wqkvT, bqkv, wo, w1, w2, vec)


def kernel(all_codes_embs, input_ids, wqkvT, bqkv, wo, w1, w2, vec):
    B, S, L = input_ids.shape
    N = B * S
    D = all_codes_embs.shape[-1]

    ids2 = input_ids.reshape(N, L)
    row_zero = jnp.all(ids2 == 0, axis=-1)
    pad = (ids2 == 0) ^ row_zero[:, None]
    keep = jnp.logical_not(pad).astype(jnp.float32)
    ev = jnp.logical_not(row_zero).astype(jnp.float32)[:, None]
    pool_w = keep * ev / keep.sum(axis=-1, keepdims=True)

    x_flat = all_codes_embs.reshape(N * L, D).astype(jnp.float32)
    pooled = _run_encoder(x_flat, keep, pool_w,
                          (wqkvT, bqkv, wo, w1, w2, vec), 8)
    return pooled.reshape(B, S, D)
